# baseline (speedup 1.0000x reference)
import jax
import jax.numpy as jnp
from jax.experimental import pallas as pl
from jax.experimental.pallas import tpu as pltpu

V = 8192
D = 1024
B = 4 * 8192
OUT_BLOCK = 1024
NBLK = B // OUT_BLOCK


def _tc_body(idx_ref, table_ref, out_ref):
    def loop(j, carry):
        out_ref[j] = table_ref[idx_ref[j]]
        return carry
    jax.lax.fori_loop(0, OUT_BLOCK, loop, 0, unroll=8)


def _tc_gather(idx, table3):
    return pl.pallas_call(
        _tc_body,
        grid=(NBLK,),
        in_specs=[
            pl.BlockSpec((OUT_BLOCK,), lambda i: (i,), memory_space=pltpu.SMEM),
            pl.BlockSpec((V, 8, 128), lambda i: (0, 0, 0)),
        ],
        out_specs=pl.BlockSpec((OUT_BLOCK, 8, 128), lambda i: (i, 0, 0)),
        out_shape=jax.ShapeDtypeStruct((B, 8, 128), jnp.float32),
        compiler_params=pltpu.CompilerParams(
            dimension_semantics=("parallel",),
        ),
    )(idx, table3)


def kernel(X, pos_embed_weight):
    flat_idx = X.reshape(-1).astype(jnp.int32)
    table3 = pos_embed_weight.reshape(V, 8, 128)
    out = _tc_gather(flat_idx, table3)
    return out.reshape(X.shape + (D,))


# speedup vs baseline: 2.0407x; 2.0407x over previous
import functools
import jax
import jax.numpy as jnp
from jax import lax
from jax.experimental import pallas as pl
from jax.experimental.pallas import tpu as pltpu
from jax.experimental.pallas import tpu_sc as plsc

V = 8192
D = 1024
B = 4 * 8192
NC = 2
NS = 16
NW = NC * NS
BPW = B // NW
K = 64
NCHUNK = BPW // K


def _sc_body(idx_hbm, table_hbm, out_hbm, idx_v, rows_v, sem):
    wid = lax.axis_index("s") * NC + lax.axis_index("c")
    pltpu.sync_copy(idx_hbm.at[wid], idx_v)

    def chunk(j, carry):
        pltpu.async_copy(table_hbm.at[idx_v.at[j]], rows_v, sem).wait()
        pltpu.sync_copy(rows_v, out_hbm.at[wid, j])
        return carry

    lax.fori_loop(0, NCHUNK, chunk, 0)


@jax.jit
def _sc_gather(idx3, table):
    mesh = plsc.VectorSubcoreMesh(core_axis_name="c", subcore_axis_name="s")
    run = pl.kernel(
        _sc_body,
        mesh=mesh,
        out_type=jax.ShapeDtypeStruct((NW, NCHUNK, K, D), jnp.float32),
        scratch_types=[
            pltpu.VMEM((NCHUNK, K), jnp.int32),
            pltpu.VMEM((K, D), jnp.float32),
            pltpu.SemaphoreType.DMA,
        ],
    )
    return run(idx3, table)


def kernel(X, pos_embed_weight):
    idx3 = X.reshape(NW, NCHUNK, K).astype(jnp.int32)
    out = _sc_gather(idx3, pos_embed_weight)
    return out.reshape(X.shape + (D,))


# speedup vs baseline: 2.1537x; 1.0553x over previous
import functools
import jax
import jax.numpy as jnp
from jax import lax
from jax.experimental import pallas as pl
from jax.experimental.pallas import tpu as pltpu
from jax.experimental.pallas import tpu_sc as plsc

V = 8192
D = 1024
B = 4 * 8192
NC = 2
NS = 16
NW = NC * NS
BPW = B // NW
K = 32
NCHUNK = BPW // K


def _sc_body(idx_hbm, table_hbm, out_hbm, idx_v, r0, r1, g0, g1, w0, w1):
    wid = lax.axis_index("s") * NC + lax.axis_index("c")
    pltpu.sync_copy(idx_hbm.at[wid], idx_v)

    bufs = (r0, r1)
    gsems = (g0, g1)
    wsems = (w0, w1)

    gathers = [None] * NCHUNK
    for j in range(2):
        gathers[j] = pltpu.async_copy(
            table_hbm.at[idx_v.at[j]], bufs[j % 2], gsems[j % 2])
    for j in range(NCHUNK):
        b = j % 2
        gathers[j].wait()
        write = pltpu.async_copy(bufs[b], out_hbm.at[wid, j], wsems[b])
        if j + 2 < NCHUNK:
            write.wait()
            gathers[j + 2] = pltpu.async_copy(
                table_hbm.at[idx_v.at[j + 2]], bufs[b], gsems[b])
        else:
            write.wait()


@jax.jit
def _sc_gather(idx3, table):
    mesh = plsc.VectorSubcoreMesh(core_axis_name="c", subcore_axis_name="s")
    run = pl.kernel(
        _sc_body,
        mesh=mesh,
        out_type=jax.ShapeDtypeStruct((NW, NCHUNK, K, D), jnp.float32),
        scratch_types=[
            pltpu.VMEM((NCHUNK, K), jnp.int32),
            pltpu.VMEM((K, D), jnp.float32),
            pltpu.VMEM((K, D), jnp.float32),
            pltpu.SemaphoreType.DMA,
            pltpu.SemaphoreType.DMA,
            pltpu.SemaphoreType.DMA,
            pltpu.SemaphoreType.DMA,
        ],
    )
    return run(idx3, table)


def kernel(X, pos_embed_weight):
    idx3 = X.reshape(NW, NCHUNK, K).astype(jnp.int32)
    out = _sc_gather(idx3, pos_embed_weight)
    return out.reshape(X.shape + (D,))
